# Initial kernel scaffold; baseline (speedup 1.0000x reference)
import jax
import jax.numpy as jnp
from jax.experimental import pallas as pl


def kernel(x, memory, src_mask, tgt_mask):
    raise NotImplementedError("write your pallas kernel here")


# speedup vs baseline: 565.7229x; 565.7229x over previous
import functools

import jax
import jax.numpy as jnp
from jax.experimental import pallas as pl


def _prior_kernel(x_ref, mem_ref, out_ref, *, n, d):
    x = x_ref[0]
    mem = mem_ref[0]
    scale = 1.0 / jnp.sqrt(jnp.float32(d))

    s = jax.lax.dot_general(x, mem, (((1,), (1,)), ((), ())),
                            preferred_element_type=jnp.float32) * scale
    smax = jnp.max(s, axis=1, keepdims=True)
    ssum = jnp.sum(jnp.exp(s - smax), axis=1, keepdims=True)
    xm_max = 1.0 / ssum

    x0 = x[0:1, :]
    s0 = jax.lax.dot_general(x, x0, (((1,), (1,)), ((), ())),
                             preferred_element_type=jnp.float32) * scale
    e0 = jnp.exp(s0 - jnp.max(s0))
    xx0 = e0 / jnp.sum(e0)

    a = xx0 + xm_max
    ridx = jax.lax.broadcasted_iota(jnp.int32, (n, 1), 0)
    a = jnp.where(ridx == 0, 0.0, a)
    amax = jnp.max(a)
    c = jnp.min(jnp.where(a == amax, ridx, n))

    lane = jax.lax.broadcasted_iota(jnp.int32, (1, n), 1)
    out = jnp.where(lane == 0, 0, c)
    out = jnp.where(lane == c, n - 1, out)
    out_ref[0] = out


def kernel(x, memory, src_mask, tgt_mask):
    b, n, d = x.shape
    m = memory.shape[1]
    out = pl.pallas_call(
        functools.partial(_prior_kernel, n=n, d=d),
        grid=(b,),
        in_specs=[
            pl.BlockSpec((1, n, d), lambda i: (i, 0, 0)),
            pl.BlockSpec((1, m, d), lambda i: (i, 0, 0)),
        ],
        out_specs=pl.BlockSpec((1, 1, n), lambda i: (i, 0, 0)),
        out_shape=jax.ShapeDtypeStruct((b, 1, n), jnp.int32),
    )(x, memory)
    return out.reshape(b, n)


# speedup vs baseline: 1083.0908x; 1.9145x over previous
import functools

import jax
import jax.numpy as jnp
from jax.experimental import pallas as pl


def _prior_kernel(x_ref, mem_ref, out_ref, *, n, d, bb):
    scale = 1.0 / jnp.sqrt(jnp.float32(d))
    ridx = jax.lax.broadcasted_iota(jnp.int32, (n, 1), 0)
    lane = jax.lax.broadcasted_iota(jnp.int32, (1, n), 1)
    for k in range(bb):
        x = x_ref[k]
        mem = mem_ref[k]

        s = jax.lax.dot_general(x, mem, (((1,), (1,)), ((), ())),
                                preferred_element_type=jnp.float32) * scale
        smax = jnp.max(s, axis=1, keepdims=True)
        ssum = jnp.sum(jnp.exp(s - smax), axis=1, keepdims=True)
        xm_max = 1.0 / ssum

        x0 = x[0:1, :]
        s0 = jax.lax.dot_general(x, x0, (((1,), (1,)), ((), ())),
                                 preferred_element_type=jnp.float32) * scale
        e0 = jnp.exp(s0 - jnp.max(s0))
        xx0 = e0 / jnp.sum(e0)

        a = xx0 + xm_max
        a = jnp.where(ridx == 0, 0.0, a)
        amax = jnp.max(a)
        c = jnp.min(jnp.where(a == amax, ridx, n))

        out = jnp.where(lane == 0, 0, c)
        out = jnp.where(lane == c, n - 1, out)
        out_ref[k] = out


def kernel(x, memory, src_mask, tgt_mask):
    b, n, d = x.shape
    m = memory.shape[1]
    bb = 4
    out = pl.pallas_call(
        functools.partial(_prior_kernel, n=n, d=d, bb=bb),
        grid=(b // bb,),
        in_specs=[
            pl.BlockSpec((bb, n, d), lambda i: (i, 0, 0)),
            pl.BlockSpec((bb, m, d), lambda i: (i, 0, 0)),
        ],
        out_specs=pl.BlockSpec((bb, 1, n), lambda i: (i, 0, 0)),
        out_shape=jax.ShapeDtypeStruct((b, 1, n), jnp.int32),
    )(x, memory)
    return out.reshape(b, n)


# speedup vs baseline: 1203.2996x; 1.1110x over previous
import functools

import jax
import jax.numpy as jnp
from jax.experimental import pallas as pl


def _prior_kernel(x_ref, mem_ref, out_ref, *, n, d, bb):
    scale = 1.0 / jnp.sqrt(jnp.float32(d))
    ridx = jax.lax.broadcasted_iota(jnp.int32, (n, 1), 0)
    lane = jax.lax.broadcasted_iota(jnp.int32, (1, n), 1)
    for k in range(bb):
        x = x_ref[k]
        mem = mem_ref[k]

        s = jax.lax.dot_general(x, mem, (((1,), (1,)), ((), ())),
                                preferred_element_type=jnp.float32) * scale
        smax = jnp.max(s, axis=1, keepdims=True)
        ssum = jnp.sum(jnp.exp(s - smax), axis=1, keepdims=True)
        xm_max = 1.0 / ssum

        x0 = x[0:1, :]
        s0 = jax.lax.dot_general(x, x0, (((1,), (1,)), ((), ())),
                                 preferred_element_type=jnp.float32) * scale
        e0 = jnp.exp(s0 - jnp.max(s0))
        xx0 = e0 / jnp.sum(e0)

        a = xx0 + xm_max
        a = jnp.where(ridx == 0, 0.0, a)
        amax = jnp.max(a)
        c = jnp.min(jnp.where(a == amax, ridx, n))

        out = jnp.where(lane == 0, 0, c)
        out = jnp.where(lane == c, n - 1, out)
        out_ref[k] = out


def kernel(x, memory, src_mask, tgt_mask):
    b, n, d = x.shape
    m = memory.shape[1]
    bb = 8
    out = pl.pallas_call(
        functools.partial(_prior_kernel, n=n, d=d, bb=bb),
        grid=(b // bb,),
        in_specs=[
            pl.BlockSpec((bb, n, d), lambda i: (i, 0, 0)),
            pl.BlockSpec((bb, m, d), lambda i: (i, 0, 0)),
        ],
        out_specs=pl.BlockSpec((bb, 1, n), lambda i: (i, 0, 0)),
        out_shape=jax.ShapeDtypeStruct((b, 1, n), jnp.int32),
    )(x, memory)
    return out.reshape(b, n)
